# baseline (speedup 1.0000x reference)
import functools
import math

import jax
import jax.numpy as jnp
from jax import lax
from jax.experimental import pallas as pl
from jax.experimental.pallas import tpu as pltpu
from jax.experimental.pallas import tpu_sc as plsc

VOCAB_SIZE = 1000000
EMB_DIM = 64
SCALE = math.sqrt(EMB_DIM)

_info = plsc.get_sparse_core_info()
NUM_CORES = _info.num_cores
NUM_SUBCORES = _info.num_subcores
LANES = _info.num_lanes
NUM_WORKERS = NUM_CORES * NUM_SUBCORES

CHUNK = 128
ET = EMB_DIM // 8
TPAD = CHUNK + 1


def _embed_kernel(n_l, idx_hbm, table_hbm, out_hbm,
                  idx_v, rows0, rows1, t0, t1, gsem0, gsem1, ssem0, ssem1):
    bt = lax.axis_index("s") * NUM_CORES + lax.axis_index("c")
    rows = (rows0, rows1)
    tout = (t0, t1)
    gsem = (gsem0, gsem1)
    ssem = (ssem0, ssem1)

    pltpu.sync_copy(idx_hbm.at[:, pl.ds(bt * CHUNK, CHUNK)], idx_v)

    def fire_gather(l, b):
        pltpu.async_copy(table_hbm.at[idx_v.at[l]], rows[b], gsem[b])

    def wait_gather(l, b):
        pltpu.make_async_copy(table_hbm.at[idx_v.at[l]], rows[b], gsem[b]).wait()

    def fire_store(l, b):
        for et in range(ET):
            pltpu.async_copy(
                tout[b].at[et, :, pl.ds(0, CHUNK)],
                out_hbm.at[l, et, bt],
                ssem[b],
            )

    def wait_store(l, b):
        for et in range(ET):
            pltpu.make_async_copy(
                tout[b].at[et, :, pl.ds(0, CHUNK)],
                out_hbm.at[l, et, bt],
                ssem[b],
            ).wait()

    fire_gather(0, 0)

    lane = lax.iota(jnp.int32, LANES)
    et_c = []
    e8_c = []
    for q in range(EMB_DIM // LANES):
        e = lane + (q * LANES)
        et_c.append(lax.shift_right_logical(e, 3))
        e8_c.append(lax.bitwise_and(e, 7))

    def step(t, _):
        for b in range(2):
            l = 2 * t + b

            @pl.when(l + 1 < n_l)
            def _():
                fire_gather(l + 1, 1 - b)

            wait_gather(l, b)

            @pl.when(l >= 2)
            def _():
                wait_store(l - 2, b)

            @plsc.parallel_loop(0, CHUNK, unroll=2)
            def _(tok):
                tok_v = jnp.full((LANES,), tok, dtype=jnp.int32)
                xs = [
                    rows[b][tok, pl.ds(q * LANES, LANES)] * SCALE
                    for q in range(EMB_DIM // LANES)
                ]
                for q in range(EMB_DIM // LANES):
                    plsc.store_scatter(
                        tout[b], [et_c[q], e8_c[q], tok_v], xs[q]
                    )

            fire_store(l, b)
        return 0

    lax.fori_loop(0, n_l // 2, step, 0)
    wait_store(n_l - 2, 0)
    wait_store(n_l - 1, 1)


N_FULL_COLS = VOCAB_SIZE // CHUNK
TAIL_ROWS = VOCAB_SIZE - N_FULL_COLS * CHUNK
N_K = (N_FULL_COLS // NUM_WORKERS) + 2


def _convert_kernel(tableT_hbm, tail_hbm, tlin_hbm, buf0, buf1, dst0, dst1,
                    gsem0, gsem1, ssem0, ssem1):
    w = lax.axis_index("s") * NUM_CORES + lax.axis_index("c")
    buf = (buf0, buf1)
    dst = (dst0, dst1)
    gsem = (gsem0, gsem1)
    ssem = (ssem0, ssem1)

    lane = lax.iota(jnp.int32, LANES)

    def transpose_col(src, dstb, n_v0):
        evecs = [lane + e0i * LANES for e0i in range(EMB_DIM // LANES)]

        @plsc.parallel_loop(0, n_v0)
        def _(v0i):
            v0 = v0i * LANES
            for j in range(LANES):
                vvec = v0 + lax.bitwise_and(lane + j, LANES - 1)
                rvec = lax.shift_right_logical(vvec, 1)
                cvec = lax.shift_left(lax.bitwise_and(vvec, 1), 6) + lane
                xs = [plsc.load_gather(src, [ev, vvec]) for ev in evecs]
                for e0i in range(EMB_DIM // LANES):
                    plsc.store_scatter(
                        dstb, [rvec, cvec + e0i * LANES], xs[e0i]
                    )

    def fire_read(c, b):
        pltpu.async_copy(
            tableT_hbm.at[:, pl.ds(c * CHUNK, CHUNK)], buf[b], gsem[b]
        )

    def wait_read(c, b):
        pltpu.make_async_copy(
            tableT_hbm.at[:, pl.ds(c * CHUNK, CHUNK)], buf[b], gsem[b]
        ).wait()

    def fire_store(c, b):
        pltpu.async_copy(dst[b], tlin_hbm.at[pl.ds(c * 64, 64), :], ssem[b])

    def drain_store(b):
        pltpu.make_async_copy(
            dst[b], tlin_hbm.at[pl.ds(0, 64), :], ssem[b]
        ).wait()

    @pl.when(w == 4)
    def _():
        pltpu.sync_copy(tail_hbm, buf[0].at[pl.ds(0, TAIL_ROWS // 2), :])
        pltpu.sync_copy(
            buf[0].at[pl.ds(0, TAIL_ROWS // 2), :],
            tlin_hbm.at[pl.ds(N_FULL_COLS * 64, TAIL_ROWS // 2), :],
        )

    fire_read(w, 0)

    def step(t, _):
        for b in range(2):
            k = 2 * t + b
            c = k * NUM_WORKERS + w

            @pl.when(c < N_FULL_COLS)
            def _():
                @pl.when(c + NUM_WORKERS < N_FULL_COLS)
                def _():
                    fire_read(c + NUM_WORKERS, 1 - b)

                wait_read(c, b)

                @pl.when(k >= 2)
                def _():
                    drain_store(b)

                transpose_col(buf[b], dst[b], CHUNK // LANES)
                fire_store(c, b)
        return 0

    lax.fori_loop(0, (N_K + 1) // 2, step, 0)
    drain_store(0)
    drain_store(1)


def kernel(tokens, table):
    B, L = tokens.shape
    assert B == NUM_WORKERS * CHUNK
    n_l = L
    assert n_l % 2 == 0

    idx = tokens.T.astype(jnp.int32)

    mesh_a = plsc.VectorSubcoreMesh(core_axis_name="c", subcore_axis_name="s")
    tlin = pl.kernel(
        _convert_kernel,
        mesh=mesh_a,
        out_type=jax.ShapeDtypeStruct((VOCAB_SIZE // 2, CHUNK), jnp.float32),
        scratch_types=[
            pltpu.VMEM((EMB_DIM, CHUNK), jnp.float32),
            pltpu.VMEM((EMB_DIM, CHUNK), jnp.float32),
            pltpu.VMEM((EMB_DIM, CHUNK), jnp.float32),
            pltpu.VMEM((EMB_DIM, CHUNK), jnp.float32),
            pltpu.SemaphoreType.DMA,
            pltpu.SemaphoreType.DMA,
            pltpu.SemaphoreType.DMA,
            pltpu.SemaphoreType.DMA,
        ],
        compiler_params=pltpu.CompilerParams(
            use_tc_tiling_on_sc=True, needs_layout_passes=False
        ),
    )(table.T, table[N_FULL_COLS * CHUNK:].reshape(TAIL_ROWS // 2, CHUNK))

    table_lin = tlin.reshape(VOCAB_SIZE, EMB_DIM)

    mesh = plsc.VectorSubcoreMesh(core_axis_name="c", subcore_axis_name="s")
    out5 = pl.kernel(
        functools.partial(_embed_kernel, n_l),
        mesh=mesh,
        out_type=jax.ShapeDtypeStruct(
            (n_l, ET, NUM_WORKERS, 8, CHUNK), jnp.float32
        ),
        scratch_types=[
            pltpu.VMEM((n_l, CHUNK), jnp.int32),
            pltpu.VMEM((CHUNK, EMB_DIM), jnp.float32),
            pltpu.VMEM((CHUNK, EMB_DIM), jnp.float32),
            pltpu.VMEM((ET, 8, TPAD), jnp.float32),
            pltpu.VMEM((ET, 8, TPAD), jnp.float32),
            pltpu.SemaphoreType.DMA,
            pltpu.SemaphoreType.DMA,
            pltpu.SemaphoreType.DMA,
            pltpu.SemaphoreType.DMA,
        ],
        compiler_params=pltpu.CompilerParams(
            use_tc_tiling_on_sc=False, needs_layout_passes=False
        ),
    )(idx, table_lin)

    out = out5.transpose(2, 4, 0, 1, 3).reshape(B, L, EMB_DIM)
    return out


# speedup vs baseline: 1.1182x; 1.1182x over previous
import functools
import math

import jax
import jax.numpy as jnp
from jax import lax
from jax.experimental import pallas as pl
from jax.experimental.pallas import tpu as pltpu
from jax.experimental.pallas import tpu_sc as plsc

VOCAB_SIZE = 1000000
EMB_DIM = 64
SCALE = math.sqrt(EMB_DIM)

_info = plsc.get_sparse_core_info()
NUM_CORES = _info.num_cores
NUM_SUBCORES = _info.num_subcores
LANES = _info.num_lanes
NUM_WORKERS = NUM_CORES * NUM_SUBCORES

CHUNK = 128
ET = EMB_DIM // 8
TPAD = CHUNK + 1


def _embed_kernel(n_l, idx_hbm, table_hbm, out_hbm,
                  idx_v, rows0, rows1, t0, t1, gsem0, gsem1, ssem0, ssem1):
    bt = lax.axis_index("s") * NUM_CORES + lax.axis_index("c")
    rows = (rows0, rows1)
    tout = (t0, t1)
    gsem = (gsem0, gsem1)
    ssem = (ssem0, ssem1)

    pltpu.sync_copy(idx_hbm.at[:, pl.ds(bt * CHUNK, CHUNK)], idx_v)

    def fire_gather(l, b):
        pltpu.async_copy(table_hbm.at[idx_v.at[l]], rows[b], gsem[b])

    def wait_gather(l, b):
        pltpu.make_async_copy(table_hbm.at[idx_v.at[l]], rows[b], gsem[b]).wait()

    def fire_store(l, b):
        for et in range(ET):
            pltpu.async_copy(
                tout[b].at[et, :, pl.ds(0, CHUNK)],
                out_hbm.at[l, et, bt],
                ssem[b],
            )

    def wait_store(l, b):
        for et in range(ET):
            pltpu.make_async_copy(
                tout[b].at[et, :, pl.ds(0, CHUNK)],
                out_hbm.at[l, et, bt],
                ssem[b],
            ).wait()

    fire_gather(0, 0)

    lane = lax.iota(jnp.int32, LANES)
    et_c = []
    e8_c = []
    for q in range(EMB_DIM // LANES):
        e = lane + (q * LANES)
        et_c.append(lax.shift_right_logical(e, 3))
        e8_c.append(lax.bitwise_and(e, 7))

    def step(t, _):
        for b in range(2):
            l = 2 * t + b

            @pl.when(l + 1 < n_l)
            def _():
                fire_gather(l + 1, 1 - b)

            wait_gather(l, b)

            @pl.when(l >= 2)
            def _():
                wait_store(l - 2, b)

            @plsc.parallel_loop(0, CHUNK, unroll=2)
            def _(tok):
                tok_v = jnp.full((LANES,), tok, dtype=jnp.int32)
                for q in range(EMB_DIM // LANES):
                    x = rows[b][tok, pl.ds(q * LANES, LANES)] * SCALE
                    plsc.store_scatter(tout[b], [et_c[q], e8_c[q], tok_v], x)

            fire_store(l, b)
        return 0

    lax.fori_loop(0, n_l // 2, step, 0)
    wait_store(n_l - 2, 0)
    wait_store(n_l - 1, 1)


N_FULL_COLS = VOCAB_SIZE // CHUNK
TAIL_ROWS = VOCAB_SIZE - N_FULL_COLS * CHUNK
N_K = (N_FULL_COLS // NUM_WORKERS) + 2


def _convert_kernel(tableT_hbm, tail_hbm, tlin_hbm, buf0, buf1, dst0, dst1,
                    gsem0, gsem1, ssem0, ssem1):
    w = lax.axis_index("s") * NUM_CORES + lax.axis_index("c")
    buf = (buf0, buf1)
    dst = (dst0, dst1)
    gsem = (gsem0, gsem1)
    ssem = (ssem0, ssem1)

    lane = lax.iota(jnp.int32, LANES)

    def transpose_col(src, dstb, n_v0):
        evecs = [lane + e0i * LANES for e0i in range(EMB_DIM // LANES)]

        @plsc.parallel_loop(0, n_v0)
        def _(v0i):
            v0 = v0i * LANES
            for j in range(LANES):
                vvec = v0 + lax.bitwise_and(lane + j, LANES - 1)
                rvec = lax.shift_right_logical(vvec, 1)
                cvec = lax.shift_left(lax.bitwise_and(vvec, 1), 6) + lane
                xs = [plsc.load_gather(src, [ev, vvec]) for ev in evecs]
                for e0i in range(EMB_DIM // LANES):
                    plsc.store_scatter(
                        dstb, [rvec, cvec + e0i * LANES], xs[e0i]
                    )

    def fire_read(c, b):
        pltpu.async_copy(
            tableT_hbm.at[:, pl.ds(c * CHUNK, CHUNK)], buf[b], gsem[b]
        )

    def wait_read(c, b):
        pltpu.make_async_copy(
            tableT_hbm.at[:, pl.ds(c * CHUNK, CHUNK)], buf[b], gsem[b]
        ).wait()

    def fire_store(c, b):
        pltpu.async_copy(dst[b], tlin_hbm.at[pl.ds(c * 64, 64), :], ssem[b])

    def drain_store(b):
        pltpu.make_async_copy(
            dst[b], tlin_hbm.at[pl.ds(0, 64), :], ssem[b]
        ).wait()

    @pl.when(w == 4)
    def _():
        pltpu.sync_copy(tail_hbm, buf[0].at[pl.ds(0, TAIL_ROWS // 2), :])
        pltpu.sync_copy(
            buf[0].at[pl.ds(0, TAIL_ROWS // 2), :],
            tlin_hbm.at[pl.ds(N_FULL_COLS * 64, TAIL_ROWS // 2), :],
        )

    fire_read(w, 0)

    def step(t, _):
        for b in range(2):
            k = 2 * t + b
            c = k * NUM_WORKERS + w

            @pl.when(c < N_FULL_COLS)
            def _():
                @pl.when(c + NUM_WORKERS < N_FULL_COLS)
                def _():
                    fire_read(c + NUM_WORKERS, 1 - b)

                wait_read(c, b)

                @pl.when(k >= 2)
                def _():
                    drain_store(b)

                transpose_col(buf[b], dst[b], CHUNK // LANES)
                fire_store(c, b)
        return 0

    lax.fori_loop(0, (N_K + 1) // 2, step, 0)
    drain_store(0)
    drain_store(1)


def kernel(tokens, table):
    B, L = tokens.shape
    assert B == NUM_WORKERS * CHUNK
    n_l = L
    assert n_l % 2 == 0

    idx = tokens.T.astype(jnp.int32)

    mesh_a = plsc.VectorSubcoreMesh(core_axis_name="c", subcore_axis_name="s")
    tlin = pl.kernel(
        _convert_kernel,
        mesh=mesh_a,
        out_type=jax.ShapeDtypeStruct((VOCAB_SIZE // 2, CHUNK), jnp.float32),
        scratch_types=[
            pltpu.VMEM((EMB_DIM, CHUNK), jnp.float32),
            pltpu.VMEM((EMB_DIM, CHUNK), jnp.float32),
            pltpu.VMEM((EMB_DIM, CHUNK), jnp.float32),
            pltpu.VMEM((EMB_DIM, CHUNK), jnp.float32),
            pltpu.SemaphoreType.DMA,
            pltpu.SemaphoreType.DMA,
            pltpu.SemaphoreType.DMA,
            pltpu.SemaphoreType.DMA,
        ],
        compiler_params=pltpu.CompilerParams(
            use_tc_tiling_on_sc=True, needs_layout_passes=False
        ),
    )(table.T, table[N_FULL_COLS * CHUNK:].reshape(TAIL_ROWS // 2, CHUNK))

    table_lin = tlin.reshape(VOCAB_SIZE, EMB_DIM)

    mesh = plsc.VectorSubcoreMesh(core_axis_name="c", subcore_axis_name="s")
    out5 = pl.kernel(
        functools.partial(_embed_kernel, n_l),
        mesh=mesh,
        out_type=jax.ShapeDtypeStruct(
            (n_l, ET, NUM_WORKERS, 8, CHUNK), jnp.float32
        ),
        scratch_types=[
            pltpu.VMEM((n_l, CHUNK), jnp.int32),
            pltpu.VMEM((CHUNK, EMB_DIM), jnp.float32),
            pltpu.VMEM((CHUNK, EMB_DIM), jnp.float32),
            pltpu.VMEM((ET, 8, TPAD), jnp.float32),
            pltpu.VMEM((ET, 8, TPAD), jnp.float32),
            pltpu.SemaphoreType.DMA,
            pltpu.SemaphoreType.DMA,
            pltpu.SemaphoreType.DMA,
            pltpu.SemaphoreType.DMA,
        ],
        compiler_params=pltpu.CompilerParams(
            use_tc_tiling_on_sc=False, needs_layout_passes=False
        ),
    )(idx, table_lin)

    out = out5.transpose(2, 4, 0, 1, 3).reshape(B, L, EMB_DIM)
    return out
